# baseline (speedup 1.0000x reference)
import functools

import jax
import jax.numpy as jnp
from jax import lax
from jax.experimental import pallas as pl
from jax.experimental.pallas import tpu as pltpu
from jax.experimental.pallas import tpu_sc as plsc

D = 64
LANES = 16
NW = 32
CHUNK = 128
NTAB = 5

_STARTS = (0, 520, 1040, 1560, 2568)
_TOTAL = 3576


def _prep_body(flag_ref, contour_ref, order_ref, arg_ref, fcw_ref, fcb_ref,
               t_ref):
    fcw = fcw_ref[...]
    n_arg = arg_ref.shape[0]
    n_big = contour_ref.shape[0]
    t_ref[_STARTS[0]:_STARTS[0] + n_arg] = lax.dot_general(
        arg_ref[...], fcw[:, :D], (((1,), (1,)), ((), ())),
        preferred_element_type=jnp.float32)
    t_ref[_STARTS[1]:_STARTS[1] + n_arg] = lax.dot_general(
        arg_ref[...], fcw[:, D:], (((1,), (1,)), ((), ())),
        preferred_element_type=jnp.float32)
    t_ref[_STARTS[2]:_STARTS[2] + flag_ref.shape[0]] = (
        flag_ref[...] + fcb_ref[...])
    t_ref[_STARTS[3]:_STARTS[3] + n_big] = contour_ref[...]
    t_ref[_STARTS[4]:_STARTS[4] + n_big] = order_ref[...]


def _prep_table(flag_w, contour_w, order_w, arg_w, fc_w, fc_b):
    return pl.pallas_call(
        _prep_body,
        out_shape=jax.ShapeDtypeStruct((_TOTAL, D), jnp.float32),
    )(flag_w, contour_w, order_w, arg_w, fc_w, fc_b.reshape(1, D))


@functools.lru_cache(maxsize=None)
def _make_sc_kernel(n_tokens):
    per_w = n_tokens // NW
    n_chunks = per_w // CHUNK
    assert per_w * NW == n_tokens and n_chunks * CHUNK == per_w
    assert n_chunks % 2 == 0
    vecs_per_row = CHUNK // LANES
    mesh = plsc.VectorSubcoreMesh(core_axis_name="c", subcore_axis_name="s")

    @functools.partial(
        pl.kernel,
        mesh=mesh,
        out_type=jax.ShapeDtypeStruct((n_tokens, D), jnp.float32),
        scratch_types=[
            pltpu.VMEM((2, NTAB, CHUNK), jnp.int32),
            pltpu.VMEM((2, NTAB, CHUNK), jnp.int32),
            pltpu.VMEM((2, NTAB * CHUNK, D), jnp.float32),
            pltpu.VMEM((2, CHUNK, D), jnp.float32),
            pltpu.SemaphoreType.DMA,
            pltpu.SemaphoreType.DMA,
        ],
        compiler_params=pltpu.CompilerParams(use_tc_tiling_on_sc=False),
    )
    def sc_fn(fonts_hbm, table_hbm, out_hbm, fbuf, idx_v, rows_v, obuf,
              sem0, sem1):
        wid = lax.axis_index("s") * 2 + lax.axis_index("c")
        sems = (sem0, sem1)
        base = wid * per_w
        lane = lax.iota(jnp.int32, LANES)

        def starts_of(t):
            v = jnp.full((LANES,), _STARTS[0] + 1, jnp.int32)
            for c in range(1, NTAB):
                v = jnp.where(t == c, _STARTS[c] + 1, v)
            return v
        off_mod = [starts_of(lax.rem(lane + m, NTAB)) for m in range(NTAB)]

        def load(g, slot):
            pltpu.sync_copy(fonts_hbm.at[wid * n_chunks + g], fbuf.at[slot])
            for r in range(NTAB):
                for v in range(vecs_per_row):
                    sl = pl.ds(v * LANES, LANES)
                    m = (r * CHUNK + v * LANES) % NTAB
                    idx_v[slot, r, sl] = fbuf[slot, r, sl] + off_mod[m]
            for r in range(NTAB):
                pltpu.make_async_copy(
                    table_hbm.at[idx_v.at[slot, r]],
                    rows_v.at[slot, pl.ds(r * CHUNK, CHUNK)],
                    sems[slot],
                ).start()

        def process(g, slot):
            @pl.when(g + 1 < n_chunks)
            def _():
                load(g + 1, 1 - slot)

            for r in range(NTAB):
                pltpu.make_async_copy(
                    table_hbm.at[idx_v.at[slot, r]],
                    rows_v.at[slot, pl.ds(r * CHUNK, CHUNK)],
                    sems[slot],
                ).wait()

            def sum_row(j, carry):
                e = j * NTAB
                for cc in range(D // LANES):
                    sl = pl.ds(cc * LANES, LANES)
                    acc = (rows_v[slot, e, sl]
                           + rows_v[slot, e + 1, sl]
                           + rows_v[slot, e + 2, sl]
                           + rows_v[slot, e + 3, sl]
                           + rows_v[slot, e + 4, sl])
                    obuf[slot, j, sl] = acc
                return carry

            lax.fori_loop(0, CHUNK, sum_row, 0)
            pltpu.sync_copy(obuf.at[slot],
                            out_hbm.at[pl.ds(base + g * CHUNK, CHUNK)])

        load(0, 0)

        def pair(p, carry):
            process(2 * p, 0)
            process(2 * p + 1, 1)
            return carry

        lax.fori_loop(0, n_chunks // 2, pair, 0)

    return sc_fn


def kernel(fonts, flag_w, contour_w, order_w, arg_w, fc_w, fc_b):
    b, s, en = fonts.shape
    n_tokens = b * s
    table = _prep_table(flag_w, contour_w, order_w, arg_w, fc_w, fc_b)
    sc_fn = _make_sc_kernel(n_tokens)
    out = sc_fn(fonts.reshape(n_tokens // CHUNK, NTAB, CHUNK), table)
    return out.reshape(b, s, D)


# speedup vs baseline: 1.4888x; 1.4888x over previous
import functools

import jax
import jax.numpy as jnp
from jax import lax
from jax.experimental import pallas as pl
from jax.experimental.pallas import tpu as pltpu
from jax.experimental.pallas import tpu_sc as plsc

D = 64
LANES = 16
NW = 32
CHUNK = 128
NTAB = 5


def _permcast(x):
    r = x.shape[0]
    y = x.reshape(r, 2, 2, LANES).transpose(0, 1, 3, 2).reshape(r, D)
    return y.astype(jnp.bfloat16)


def _prep_body(flag_ref, contour_ref, order_ref, arg_ref, fcw_ref, fcb_ref,
               a0_ref, a1_ref, flagb_ref, cont_ref, ord_ref):
    fcw = fcw_ref[...]
    a0_ref[...] = _permcast(lax.dot_general(
        arg_ref[...], fcw[:, :D], (((1,), (1,)), ((), ())),
        preferred_element_type=jnp.float32))
    a1_ref[...] = _permcast(lax.dot_general(
        arg_ref[...], fcw[:, D:], (((1,), (1,)), ((), ())),
        preferred_element_type=jnp.float32))
    flagb_ref[...] = _permcast(flag_ref[...] + fcb_ref[...])
    cont_ref[...] = _permcast(contour_ref[...])
    ord_ref[...] = _permcast(order_ref[...])


def _prep_tables(flag_w, contour_w, order_w, arg_w, fc_w, fc_b):
    n_arg = arg_w.shape[0]
    return pl.pallas_call(
        _prep_body,
        out_shape=[
            jax.ShapeDtypeStruct((n_arg, D), jnp.bfloat16),
            jax.ShapeDtypeStruct((n_arg, D), jnp.bfloat16),
            jax.ShapeDtypeStruct((flag_w.shape[0], D), jnp.bfloat16),
            jax.ShapeDtypeStruct((contour_w.shape[0], D), jnp.bfloat16),
            jax.ShapeDtypeStruct((order_w.shape[0], D), jnp.bfloat16),
        ],
    )(flag_w, contour_w, order_w, arg_w, fc_w, fc_b.reshape(1, D))


@functools.lru_cache(maxsize=None)
def _make_sc_kernel(n_tokens):
    per_w = n_tokens // NW
    n_chunks = per_w // CHUNK
    assert per_w * NW == n_tokens and n_chunks * CHUNK == per_w
    assert n_chunks % 2 == 0
    mesh = plsc.VectorSubcoreMesh(core_axis_name="c", subcore_axis_name="s")

    @functools.partial(
        pl.kernel,
        mesh=mesh,
        out_type=jax.ShapeDtypeStruct((n_tokens, D), jnp.float32),
        scratch_types=[
            pltpu.VMEM((2, NTAB, CHUNK), jnp.int32),
            pltpu.VMEM((2, NTAB, CHUNK, D), jnp.bfloat16),
            pltpu.VMEM((2, CHUNK, D), jnp.float32),
            pltpu.SemaphoreType.DMA,
            pltpu.SemaphoreType.DMA,
        ],
        compiler_params=pltpu.CompilerParams(use_tc_tiling_on_sc=False,
                                             needs_layout_passes=False),
    )
    def sc_fn(idx_hbm, t0, t1, t2, t3, t4, out_hbm, idx_v, rows_v, obuf,
              sem0, sem1):
        wid = lax.axis_index("s") * 2 + lax.axis_index("c")
        tables = (t0, t1, t2, t3, t4)
        sems = (sem0, sem1)
        base = wid * per_w

        def load(g, slot):
            pltpu.sync_copy(idx_hbm.at[wid, g], idx_v.at[slot])
            for c in range(NTAB):
                pltpu.make_async_copy(
                    tables[c].at[idx_v.at[slot, c]],
                    rows_v.at[slot, c],
                    sems[slot],
                ).start()

        def process(g, slot):
            @pl.when(g + 1 < n_chunks)
            def _():
                load(g + 1, 1 - slot)

            for c in range(NTAB):
                pltpu.make_async_copy(
                    tables[c].at[idx_v.at[slot, c]],
                    rows_v.at[slot, c],
                    sems[slot],
                ).wait()

            def sum_row(r, carry):
                for cc in range(2):
                    sl = pl.ds(cc * 2 * LANES, 2 * LANES)
                    acc = (rows_v[slot, 0, r, sl]
                           + rows_v[slot, 1, r, sl]
                           + rows_v[slot, 2, r, sl]
                           + rows_v[slot, 3, r, sl]
                           + rows_v[slot, 4, r, sl])
                    lo, hi = plsc.unpack(
                        acc, format=plsc.PackFormat.INTERLEAVED)
                    obuf[slot, r, pl.ds(cc * 2 * LANES, LANES)] = lo
                    obuf[slot, r, pl.ds(cc * 2 * LANES + LANES, LANES)] = hi
                return carry

            lax.fori_loop(0, CHUNK, sum_row, 0)
            pltpu.sync_copy(obuf.at[slot],
                            out_hbm.at[pl.ds(base + g * CHUNK, CHUNK)])

        load(0, 0)

        def pair(p, carry):
            process(2 * p, 0)
            process(2 * p + 1, 1)
            return carry

        lax.fori_loop(0, n_chunks // 2, pair, 0)

    return sc_fn


def kernel(fonts, flag_w, contour_w, order_w, arg_w, fc_w, fc_b):
    b, s, en = fonts.shape
    n_tokens = b * s
    a0, a1, flagb, cont, ordr = _prep_tables(
        flag_w, contour_w, order_w, arg_w, fc_w, fc_b)
    f = fonts.reshape(NW, n_tokens // (NW * CHUNK), CHUNK, en) + 1
    idx = f.transpose(0, 1, 3, 2)
    sc_fn = _make_sc_kernel(n_tokens)
    out = sc_fn(idx, a0, a1, flagb, cont, ordr)
    return out.reshape(b, s, D)


# speedup vs baseline: 1.7589x; 1.1814x over previous
import functools

import jax
import jax.numpy as jnp
from jax import lax
from jax.experimental import pallas as pl
from jax.experimental.pallas import tpu as pltpu
from jax.experimental.pallas import tpu_sc as plsc

D = 64
LANES = 16
NW = 32
CHUNK = 128
NTAB = 5


def _permcast(x):
    r = x.shape[0]
    y = x.reshape(r, 2, 2, LANES).transpose(0, 1, 3, 2).reshape(r, D)
    return y.astype(jnp.bfloat16)


def _prep_body(flag_ref, contour_ref, order_ref, arg_ref, fcw_ref, fcb_ref,
               a0_ref, a1_ref, flagb_ref, cont_ref, ord_ref):
    fcw = fcw_ref[...]
    a0_ref[...] = _permcast(lax.dot_general(
        arg_ref[...], fcw[:, :D], (((1,), (1,)), ((), ())),
        preferred_element_type=jnp.float32))
    a1_ref[...] = _permcast(lax.dot_general(
        arg_ref[...], fcw[:, D:], (((1,), (1,)), ((), ())),
        preferred_element_type=jnp.float32))
    flagb_ref[...] = _permcast(flag_ref[...] + fcb_ref[...])
    cont_ref[...] = _permcast(contour_ref[...])
    ord_ref[...] = _permcast(order_ref[...])


def _prep_tables(flag_w, contour_w, order_w, arg_w, fc_w, fc_b):
    n_arg = arg_w.shape[0]
    return pl.pallas_call(
        _prep_body,
        out_shape=[
            jax.ShapeDtypeStruct((n_arg, D), jnp.bfloat16),
            jax.ShapeDtypeStruct((n_arg, D), jnp.bfloat16),
            jax.ShapeDtypeStruct((flag_w.shape[0], D), jnp.bfloat16),
            jax.ShapeDtypeStruct((contour_w.shape[0], D), jnp.bfloat16),
            jax.ShapeDtypeStruct((order_w.shape[0], D), jnp.bfloat16),
        ],
    )(flag_w, contour_w, order_w, arg_w, fc_w, fc_b.reshape(1, D))


@functools.lru_cache(maxsize=None)
def _make_sc_kernel(n_tokens):
    per_w = n_tokens // NW
    n_chunks = per_w // CHUNK
    assert per_w * NW == n_tokens and n_chunks * CHUNK == per_w
    assert n_chunks % 2 == 0
    mesh = plsc.VectorSubcoreMesh(core_axis_name="c", subcore_axis_name="s")

    @functools.partial(
        pl.kernel,
        mesh=mesh,
        out_type=jax.ShapeDtypeStruct((n_tokens, D), jnp.float32),
        scratch_types=[
            pltpu.VMEM((2, NTAB, CHUNK), jnp.int32),
            pltpu.VMEM((2, NTAB, CHUNK, D), jnp.bfloat16),
            pltpu.VMEM((2, CHUNK, D), jnp.float32),
            pltpu.SemaphoreType.DMA,
            pltpu.SemaphoreType.DMA,
        ],
        compiler_params=pltpu.CompilerParams(use_tc_tiling_on_sc=False,
                                             needs_layout_passes=False),
    )
    def sc_fn(idx_hbm, t0, t1, t2, t3, t4, out_hbm, idx_v, rows_v, obuf,
              sem0, sem1):
        wid = lax.axis_index("s") * 2 + lax.axis_index("c")
        tables = (t0, t1, t2, t3, t4)
        sems = (sem0, sem1)
        base = wid * per_w

        def load(g, slot):
            pltpu.sync_copy(idx_hbm.at[wid, g], idx_v.at[slot])
            for c in range(NTAB):
                pltpu.make_async_copy(
                    tables[c].at[idx_v.at[slot, c]],
                    rows_v.at[slot, c],
                    sems[slot],
                ).start()

        def process(g, slot):
            @pl.when(g + 1 < n_chunks)
            def _():
                load(g + 1, 1 - slot)

            for c in range(NTAB):
                pltpu.make_async_copy(
                    tables[c].at[idx_v.at[slot, c]],
                    rows_v.at[slot, c],
                    sems[slot],
                ).wait()

            @plsc.parallel_loop(0, CHUNK, step=4, unroll=2)
            def _(r):
                for dr in range(4):
                    rr = r + dr
                    for cc in range(2):
                        sl = pl.ds(cc * 2 * LANES, 2 * LANES)
                        acc = (rows_v[slot, 0, rr, sl]
                               + rows_v[slot, 1, rr, sl]
                               + rows_v[slot, 2, rr, sl]
                               + rows_v[slot, 3, rr, sl]
                               + rows_v[slot, 4, rr, sl])
                        lo, hi = plsc.unpack(
                            acc, format=plsc.PackFormat.INTERLEAVED)
                        obuf[slot, rr, pl.ds(cc * 2 * LANES, LANES)] = lo
                        obuf[slot, rr,
                             pl.ds(cc * 2 * LANES + LANES, LANES)] = hi
            pltpu.sync_copy(obuf.at[slot],
                            out_hbm.at[pl.ds(base + g * CHUNK, CHUNK)])

        load(0, 0)

        def pair(p, carry):
            process(2 * p, 0)
            process(2 * p + 1, 1)
            return carry

        lax.fori_loop(0, n_chunks // 2, pair, 0)

    return sc_fn


def kernel(fonts, flag_w, contour_w, order_w, arg_w, fc_w, fc_b):
    b, s, en = fonts.shape
    n_tokens = b * s
    a0, a1, flagb, cont, ordr = _prep_tables(
        flag_w, contour_w, order_w, arg_w, fc_w, fc_b)
    f = fonts.reshape(NW, n_tokens // (NW * CHUNK), CHUNK, en) + 1
    idx = f.transpose(0, 1, 3, 2)
    sc_fn = _make_sc_kernel(n_tokens)
    out = sc_fn(idx, a0, a1, flagb, cont, ordr)
    return out.reshape(b, s, D)
